# baseline (speedup 1.0000x reference)
import functools

import jax
import jax.numpy as jnp
from jax import lax
from jax.experimental import pallas as pl
from jax.experimental.pallas import tpu as pltpu
from jax.experimental.pallas import tpu_sc as plsc

NCORE = 2
NSUB = 16
NW = NCORE * NSUB
CH = 128


@functools.lru_cache(None)
def _deg_kernel(np_, ep):
    ew = ep // NW
    nch = ew // CH
    rps = np_ // NSUB
    zr = 64
    mesh = plsc.VectorSubcoreMesh(core_axis_name="c", subcore_axis_name="s")

    assert nch % 4 == 0
    ng = nch // 2

    def body(dst_hbm, out_hbm, dst0, dst1, dst2, dst3, ones_v, zb_v, z_sh,
             sd0, sd1, sd2, sd3, sc0, sc1):
        c = lax.axis_index("c")
        s = lax.axis_index("s")
        dst_b = (dst0, dst1, dst2, dst3)
        sd = (sd0, sd1, sd2, sd3)
        one16 = jnp.ones((16,), jnp.float32)
        zero16 = jnp.zeros((16,), jnp.float32)
        for j in range(CH // 16):
            ones_v[pl.ds(j * 16, 16)] = one16
        for j in range(zr // 16):
            zb_v[pl.ds(j * 16, 16)] = zero16
        base = s * rps

        def zbody(i, carry):
            pltpu.sync_copy(zb_v, z_sh.at[pl.ds(base + i * zr, zr)])
            return carry

        lax.fori_loop(0, rps // zr, zbody, 0)
        plsc.subcore_barrier()
        ebase = (c * NSUB + s) * ew
        last = ebase + (nch - 1) * CH

        def idx_off(m):
            return jnp.minimum(ebase + m * CH, last)

        pltpu.sync_copy(dst_hbm.at[pl.ds(ebase, CH)], dst0)
        pltpu.sync_copy(dst_hbm.at[pl.ds(ebase + CH, CH)], dst1)
        pltpu.async_copy(dst_hbm.at[pl.ds(ebase + 2 * CH, CH)], dst2, sd2)
        pltpu.async_copy(dst_hbm.at[pl.ds(ebase + 3 * CH, CH)], dst3, sd3)

        def step(g, p):
            q = 1 - p
            a, b = 2 * p, 2 * p + 1
            e0, e1 = 2 * q, 2 * q + 1
            o2, o3 = idx_off(2 * g + 2), idx_off(2 * g + 3)
            pltpu.make_async_copy(dst_hbm.at[pl.ds(o2, CH)], dst_b[e0], sd[e0]).wait()
            pltpu.make_async_copy(dst_hbm.at[pl.ds(o3, CH)], dst_b[e1], sd[e1]).wait()
            pltpu.async_copy(ones_v, z_sh.at[dst_b[a]], sc0, add=True)
            pltpu.async_copy(ones_v, z_sh.at[dst_b[b]], sc1, add=True)
            pltpu.make_async_copy(ones_v, z_sh.at[dst_b[a]], sc0).wait()
            pltpu.make_async_copy(ones_v, z_sh.at[dst_b[b]], sc1).wait()
            o4, o5 = idx_off(2 * g + 4), idx_off(2 * g + 5)
            pltpu.async_copy(dst_hbm.at[pl.ds(o4, CH)], dst_b[a], sd[a])
            pltpu.async_copy(dst_hbm.at[pl.ds(o5, CH)], dst_b[b], sd[b])

        def ebody(gg, carry):
            step(2 * gg, 0)
            step(2 * gg + 1, 1)
            return carry

        lax.fori_loop(0, ng // 2, ebody, 0)
        p_last = (ng - 1) & 1
        for i in (2 * p_last, 2 * p_last + 1):
            pltpu.make_async_copy(dst_hbm.at[pl.ds(last, CH)], dst_b[i], sd[i]).wait()
        plsc.subcore_barrier()
        pltpu.sync_copy(z_sh.at[pl.ds(base, rps)],
                        out_hbm.at[pl.ds(c * np_ + base, rps)])

    return pl.kernel(
        body,
        out_type=jax.ShapeDtypeStruct((NCORE * np_,), jnp.float32),
        mesh=mesh,
        compiler_params=pltpu.CompilerParams(use_tc_tiling_on_sc=False),
        scratch_types=(
            [pltpu.VMEM((CH,), jnp.int32)] * 4
            + [pltpu.VMEM((CH,), jnp.float32),
               pltpu.VMEM((zr,), jnp.float32),
               pltpu.VMEM_SHARED((np_,), jnp.float32)]
            + [pltpu.SemaphoreType.DMA] * 6
        ),
    )


@functools.lru_cache(None)
def _agg_kernel(np_, ep, h):
    ew = ep // NW
    nch = ew // CH
    assert nch % 4 == 0
    ng = nch // 2
    rps = np_ // NSUB
    zr = 128
    assert rps % zr == 0
    mesh = plsc.VectorSubcoreMesh(core_axis_name="c", subcore_axis_name="s")

    def body(y_hbm, src_hbm, dst_hbm, out_hbm,
             src0, src1, src2, src3, dst0, dst1, dst2, dst3,
             rows0, rows1, rows2, rows3, zb_v, z_sh, y_sh,
             sg0, sg1, sg2, sg3, ss0, ss1, ss2, ss3,
             sd0, sd1, sd2, sd3, sc0, sc1):
        c = lax.axis_index("c")
        s = lax.axis_index("s")
        src_b = (src0, src1, src2, src3)
        dst_b = (dst0, dst1, dst2, dst3)
        rows_b = (rows0, rows1, rows2, rows3)
        sg = (sg0, sg1, sg2, sg3)
        ss = (ss0, ss1, ss2, ss3)
        sd = (sd0, sd1, sd2, sd3)
        zero16 = jnp.zeros((16,), jnp.float32)
        for r in range(zr):
            for j in range(h // 16):
                zb_v[r, pl.ds(j * 16, 16)] = zero16
        base = s * rps

        def zbody(i, carry):
            pltpu.sync_copy(zb_v, z_sh.at[pl.ds(base + i * zr, zr)])
            return carry

        lax.fori_loop(0, rps // zr, zbody, 0)
        pltpu.sync_copy(y_hbm.at[pl.ds(base, rps)], y_sh.at[pl.ds(base, rps)])
        plsc.subcore_barrier()
        ebase = (c * NSUB + s) * ew
        last = ebase + (nch - 1) * CH

        def idx_off(m):
            return jnp.minimum(ebase + m * CH, last)

        pltpu.sync_copy(src_hbm.at[pl.ds(ebase, CH)], src0)
        pltpu.sync_copy(dst_hbm.at[pl.ds(ebase, CH)], dst0)
        pltpu.sync_copy(src_hbm.at[pl.ds(ebase + CH, CH)], src1)
        pltpu.sync_copy(dst_hbm.at[pl.ds(ebase + CH, CH)], dst1)
        pltpu.async_copy(y_sh.at[src0], rows0, sg0)
        pltpu.async_copy(y_sh.at[src1], rows1, sg1)
        pltpu.async_copy(src_hbm.at[pl.ds(ebase + 2 * CH, CH)], src2, ss2)
        pltpu.async_copy(dst_hbm.at[pl.ds(ebase + 2 * CH, CH)], dst2, sd2)
        pltpu.async_copy(src_hbm.at[pl.ds(ebase + 3 * CH, CH)], src3, ss3)
        pltpu.async_copy(dst_hbm.at[pl.ds(ebase + 3 * CH, CH)], dst3, sd3)

        def step(g, p):
            q = 1 - p
            a, b = 2 * p, 2 * p + 1
            e0, e1 = 2 * q, 2 * q + 1
            pltpu.make_async_copy(y_sh.at[src_b[a]], rows_b[a], sg[a]).wait()
            pltpu.make_async_copy(y_sh.at[src_b[b]], rows_b[b], sg[b]).wait()
            o2, o3 = idx_off(2 * g + 2), idx_off(2 * g + 3)
            pltpu.make_async_copy(src_hbm.at[pl.ds(o2, CH)], src_b[e0], ss[e0]).wait()
            pltpu.make_async_copy(dst_hbm.at[pl.ds(o2, CH)], dst_b[e0], sd[e0]).wait()
            pltpu.make_async_copy(src_hbm.at[pl.ds(o3, CH)], src_b[e1], ss[e1]).wait()
            pltpu.make_async_copy(dst_hbm.at[pl.ds(o3, CH)], dst_b[e1], sd[e1]).wait()
            pltpu.async_copy(y_sh.at[src_b[e0]], rows_b[e0], sg[e0])
            pltpu.async_copy(y_sh.at[src_b[e1]], rows_b[e1], sg[e1])
            pltpu.async_copy(rows_b[a], z_sh.at[dst_b[a]], sc0, add=True)
            pltpu.async_copy(rows_b[b], z_sh.at[dst_b[b]], sc1, add=True)
            pltpu.make_async_copy(rows_b[a], z_sh.at[dst_b[a]], sc0).wait()
            pltpu.make_async_copy(rows_b[b], z_sh.at[dst_b[b]], sc1).wait()
            o4, o5 = idx_off(2 * g + 4), idx_off(2 * g + 5)
            pltpu.async_copy(src_hbm.at[pl.ds(o4, CH)], src_b[a], ss[a])
            pltpu.async_copy(dst_hbm.at[pl.ds(o4, CH)], dst_b[a], sd[a])
            pltpu.async_copy(src_hbm.at[pl.ds(o5, CH)], src_b[b], ss[b])
            pltpu.async_copy(dst_hbm.at[pl.ds(o5, CH)], dst_b[b], sd[b])

        def ebody(gg, carry):
            step(2 * gg, 0)
            step(2 * gg + 1, 1)
            return carry

        lax.fori_loop(0, ng // 2, ebody, 0)
        p_last = (ng - 1) & 1
        q_last = 1 - p_last
        for i in (2 * q_last, 2 * q_last + 1):
            pltpu.make_async_copy(y_sh.at[src_b[i]], rows_b[i], sg[i]).wait()
        for i in (2 * p_last, 2 * p_last + 1):
            pltpu.make_async_copy(src_hbm.at[pl.ds(last, CH)], src_b[i], ss[i]).wait()
            pltpu.make_async_copy(dst_hbm.at[pl.ds(last, CH)], dst_b[i], sd[i]).wait()
        plsc.subcore_barrier()
        pltpu.sync_copy(z_sh.at[pl.ds(base, rps)],
                        out_hbm.at[pl.ds(c * np_ + base, rps)])

    return pl.kernel(
        body,
        out_type=jax.ShapeDtypeStruct((NCORE * np_, h), jnp.float32),
        mesh=mesh,
        compiler_params=pltpu.CompilerParams(use_tc_tiling_on_sc=False),
        scratch_types=(
            [pltpu.VMEM((CH,), jnp.int32)] * 8
            + [pltpu.VMEM((CH, h), jnp.float32)] * 4
            + [pltpu.VMEM((zr, h), jnp.float32),
               pltpu.VMEM_SHARED((np_, h), jnp.float32),
               pltpu.VMEM_SHARED((np_, h), jnp.float32)]
            + [pltpu.SemaphoreType.DMA] * 14
        ),
    )


@functools.lru_cache(None)
def _dinv_kernel(np_):
    def body(degp_ref, out_ref):
        deg = degp_ref[0, :] + degp_ref[1, :] + 1.0
        out_ref[...] = lax.rsqrt(deg)

    return pl.pallas_call(
        body, out_shape=jax.ShapeDtypeStruct((np_,), jnp.float32))


@functools.lru_cache(None)
def _first_tc(n, np_, f_in, h1):
    r = 1000

    def body(x_ref, w_ref, dinv_ref, out_ref):
        xw = jnp.dot(x_ref[...], w_ref[...],
                     preferred_element_type=jnp.float32,
                     precision=lax.Precision.HIGHEST)
        out_ref[...] = xw * dinv_ref[...]

    return pl.pallas_call(
        body,
        grid=(n // r,),
        in_specs=[
            pl.BlockSpec((r, f_in), lambda i: (i, 0)),
            pl.BlockSpec((f_in, h1), lambda i: (0, 0)),
            pl.BlockSpec((r, 1), lambda i: (i, 0)),
        ],
        out_specs=pl.BlockSpec((r, h1), lambda i: (i, 0)),
        out_shape=jax.ShapeDtypeStruct((np_, h1), jnp.float32),
    )


@functools.lru_cache(None)
def _mid_tc(np_, hin, hout):
    r = 1024

    def body(z0_ref, z1_ref, y_ref, b_ref, w_ref, dinv_ref, out_ref):
        dinv = dinv_ref[...]
        agg = z0_ref[...] + z1_ref[...] + y_ref[...]
        hcur = jnp.maximum(agg * dinv + b_ref[...], 0.0)
        out_ref[...] = jnp.dot(hcur, w_ref[...],
                               preferred_element_type=jnp.float32,
                               precision=lax.Precision.HIGHEST) * dinv

    nb = np_ // r
    return pl.pallas_call(
        body,
        grid=(nb,),
        in_specs=[
            pl.BlockSpec((r, hin), lambda i: (i, 0)),
            pl.BlockSpec((r, hin), lambda i: (i + nb, 0)),
            pl.BlockSpec((r, hin), lambda i: (i, 0)),
            pl.BlockSpec((1, hin), lambda i: (0, 0)),
            pl.BlockSpec((hin, hout), lambda i: (0, 0)),
            pl.BlockSpec((r, 1), lambda i: (i, 0)),
        ],
        out_specs=pl.BlockSpec((r, hout), lambda i: (i, 0)),
        out_shape=jax.ShapeDtypeStruct((np_, hout), jnp.float32),
    )


@functools.lru_cache(None)
def _final_tc(n, np_, hp, c_out):
    r = 80

    def body(z0_ref, z1_ref, y_ref, b_ref, dinv_ref, out_ref):
        agg = z0_ref[...] + z1_ref[...] + y_ref[...]
        res = agg * dinv_ref[...] + b_ref[...]
        out_ref[...] = res[:, :c_out]

    nb1 = np_ // r
    return pl.pallas_call(
        body,
        grid=(n // r,),
        in_specs=[
            pl.BlockSpec((r, hp), lambda i: (i, 0)),
            pl.BlockSpec((r, hp), lambda i: (i + nb1, 0)),
            pl.BlockSpec((r, hp), lambda i: (i, 0)),
            pl.BlockSpec((1, hp), lambda i: (0, 0)),
            pl.BlockSpec((r, 1), lambda i: (i, 0)),
        ],
        out_specs=pl.BlockSpec((r, c_out), lambda i: (i, 0)),
        out_shape=jax.ShapeDtypeStruct((n, c_out), jnp.float32),
    )


def kernel(x, edge_index, W1, b1, W2, b2, W3, b3):
    n, f_in = x.shape
    e = edge_index.shape[1]
    h1, h2, c_out = W1.shape[1], W2.shape[1], W3.shape[1]
    hp = -(-c_out // 16) * 16

    np_ = (n // 512 + 1) * 512
    ep = -(-e // (NW * CH * 4)) * (NW * CH * 4)

    src = edge_index[0]
    dst = edge_index[1]
    pad = ep - e
    if pad:
        fill = jnp.full((pad,), n, dtype=src.dtype)
        src = jnp.concatenate([src, fill])
        dst = jnp.concatenate([dst, fill])
    w3p = jnp.pad(W3, ((0, 0), (0, hp - c_out)))
    b3p = jnp.pad(b3, (0, hp - c_out)).reshape(1, hp)

    degp = _deg_kernel(np_, ep)(dst)
    dinv2 = _dinv_kernel(np_)(degp.reshape(NCORE, np_)).reshape(np_, 1)

    y1 = _first_tc(n, np_, f_in, h1)(x, W1, dinv2)
    z1 = _agg_kernel(np_, ep, h1)(y1, src, dst)
    y2 = _mid_tc(np_, h1, h2)(z1, z1, y1, b1.reshape(1, h1), W2, dinv2)
    z2 = _agg_kernel(np_, ep, h2)(y2, src, dst)
    y3 = _mid_tc(np_, h2, hp)(z2, z2, y2, b2.reshape(1, h2), w3p, dinv2)
    z3 = _agg_kernel(np_, ep, hp)(y3, src, dst)
    return _final_tc(n, np_, hp, c_out)(z3, z3, y3, b3p, dinv2)


# speedup vs baseline: 1.0708x; 1.0708x over previous
import functools

import jax
import jax.numpy as jnp
from jax import lax
from jax.experimental import pallas as pl
from jax.experimental.pallas import tpu as pltpu
from jax.experimental.pallas import tpu_sc as plsc

NCORE = 2
NSUB = 16
NW = NCORE * NSUB
CH = 128


@functools.lru_cache(None)
def _deg_kernel(np_, ep):
    ew = ep // NW
    nch = ew // CH
    rps = np_ // NSUB
    zr = 64
    mesh = plsc.VectorSubcoreMesh(core_axis_name="c", subcore_axis_name="s")

    assert nch % 4 == 0
    ng = nch // 2

    def body(dst_hbm, out_hbm, dst0, dst1, dst2, dst3, ones_v, zb_v, z_sh,
             sd0, sd1, sd2, sd3, sc0, sc1):
        c = lax.axis_index("c")
        s = lax.axis_index("s")
        dst_b = (dst0, dst1, dst2, dst3)
        sd = (sd0, sd1, sd2, sd3)
        one16 = jnp.ones((16,), jnp.float32)
        zero16 = jnp.zeros((16,), jnp.float32)
        for j in range(CH // 16):
            ones_v[pl.ds(j * 16, 16)] = one16
        for j in range(zr // 16):
            zb_v[pl.ds(j * 16, 16)] = zero16
        base = s * rps

        def zbody(i, carry):
            pltpu.sync_copy(zb_v, z_sh.at[pl.ds(base + i * zr, zr)])
            return carry

        lax.fori_loop(0, rps // zr, zbody, 0)
        plsc.subcore_barrier()
        ebase = (c * NSUB + s) * ew
        last = ebase + (nch - 1) * CH

        def idx_off(m):
            return jnp.minimum(ebase + m * CH, last)

        pltpu.sync_copy(dst_hbm.at[pl.ds(ebase, CH)], dst0)
        pltpu.sync_copy(dst_hbm.at[pl.ds(ebase + CH, CH)], dst1)
        pltpu.async_copy(dst_hbm.at[pl.ds(ebase + 2 * CH, CH)], dst2, sd2)
        pltpu.async_copy(dst_hbm.at[pl.ds(ebase + 3 * CH, CH)], dst3, sd3)

        def step(g, p):
            q = 1 - p
            a, b = 2 * p, 2 * p + 1
            e0, e1 = 2 * q, 2 * q + 1
            o2, o3 = idx_off(2 * g + 2), idx_off(2 * g + 3)
            pltpu.make_async_copy(dst_hbm.at[pl.ds(o2, CH)], dst_b[e0], sd[e0]).wait()
            pltpu.make_async_copy(dst_hbm.at[pl.ds(o3, CH)], dst_b[e1], sd[e1]).wait()
            pltpu.async_copy(ones_v, z_sh.at[dst_b[a]], sc0, add=True)
            pltpu.async_copy(ones_v, z_sh.at[dst_b[b]], sc1, add=True)
            pltpu.make_async_copy(ones_v, z_sh.at[dst_b[a]], sc0).wait()
            pltpu.make_async_copy(ones_v, z_sh.at[dst_b[b]], sc1).wait()
            o4, o5 = idx_off(2 * g + 4), idx_off(2 * g + 5)
            pltpu.async_copy(dst_hbm.at[pl.ds(o4, CH)], dst_b[a], sd[a])
            pltpu.async_copy(dst_hbm.at[pl.ds(o5, CH)], dst_b[b], sd[b])

        def ebody(gg, carry):
            step(2 * gg, 0)
            step(2 * gg + 1, 1)
            return carry

        lax.fori_loop(0, ng // 2, ebody, 0)
        p_last = (ng - 1) & 1
        for i in (2 * p_last, 2 * p_last + 1):
            pltpu.make_async_copy(dst_hbm.at[pl.ds(last, CH)], dst_b[i], sd[i]).wait()
        plsc.subcore_barrier()
        pltpu.sync_copy(z_sh.at[pl.ds(base, rps)],
                        out_hbm.at[pl.ds(c * np_ + base, rps)])

    return pl.kernel(
        body,
        out_type=jax.ShapeDtypeStruct((NCORE * np_,), jnp.float32),
        mesh=mesh,
        compiler_params=pltpu.CompilerParams(use_tc_tiling_on_sc=False),
        scratch_types=(
            [pltpu.VMEM((CH,), jnp.int32)] * 4
            + [pltpu.VMEM((CH,), jnp.float32),
               pltpu.VMEM((zr,), jnp.float32),
               pltpu.VMEM_SHARED((np_,), jnp.float32)]
            + [pltpu.SemaphoreType.DMA] * 6
        ),
    )


@functools.lru_cache(None)
def _agg_kernel(np_, ep, h):
    ew = ep // NW
    nch = ew // CH
    assert nch % 4 == 0
    ng = nch // 2
    rps = np_ // NSUB
    zr = 128
    assert rps % zr == 0
    mesh = plsc.VectorSubcoreMesh(core_axis_name="c", subcore_axis_name="s")

    assert rps % CH == 0

    def body(y_hbm, dinv_hbm, src_hbm, dst_hbm, out_hbm,
             src0, src1, src2, src3, dst0, dst1, dst2, dst3,
             rows0, rows1, rows2, rows3, zb_v, dbuf, z_sh, y_sh,
             sg0, sg1, sg2, sg3, ss0, ss1, ss2, ss3,
             sd0, sd1, sd2, sd3, sc0, sc1):
        c = lax.axis_index("c")
        s = lax.axis_index("s")
        src_b = (src0, src1, src2, src3)
        dst_b = (dst0, dst1, dst2, dst3)
        rows_b = (rows0, rows1, rows2, rows3)
        sg = (sg0, sg1, sg2, sg3)
        ss = (ss0, ss1, ss2, ss3)
        sd = (sd0, sd1, sd2, sd3)
        zero16 = jnp.zeros((16,), jnp.float32)
        for r in range(zr):
            for j in range(h // 16):
                zb_v[r, pl.ds(j * 16, 16)] = zero16
        base = s * rps

        def scale_rows(buf):
            def rloop(i, carry):
                dv = dbuf[pl.ds(i * 16, 16)]
                for rr in range(16):
                    r = i * 16 + rr
                    sv = dv[rr]
                    for j in range(h // 16):
                        buf[r, pl.ds(16 * j, 16)] = buf[r, pl.ds(16 * j, 16)] * sv
                return carry
            lax.fori_loop(0, CH // 16, rloop, 0)

        def zbody(i, carry):
            pltpu.sync_copy(zb_v, z_sh.at[pl.ds(base + i * zr, zr)])
            return carry

        lax.fori_loop(0, rps // zr, zbody, 0)
        for t in range(rps // CH):
            rb = base + t * CH
            pltpu.sync_copy(y_hbm.at[pl.ds(rb, CH)], rows0)
            pltpu.sync_copy(dinv_hbm.at[pl.ds(rb, CH)], dbuf)
            scale_rows(rows0)
            pltpu.sync_copy(rows0, y_sh.at[pl.ds(rb, CH)])
        plsc.subcore_barrier()
        ebase = (c * NSUB + s) * ew
        last = ebase + (nch - 1) * CH

        def idx_off(m):
            return jnp.minimum(ebase + m * CH, last)

        pltpu.sync_copy(src_hbm.at[pl.ds(ebase, CH)], src0)
        pltpu.sync_copy(dst_hbm.at[pl.ds(ebase, CH)], dst0)
        pltpu.sync_copy(src_hbm.at[pl.ds(ebase + CH, CH)], src1)
        pltpu.sync_copy(dst_hbm.at[pl.ds(ebase + CH, CH)], dst1)
        pltpu.async_copy(y_sh.at[src0], rows0, sg0)
        pltpu.async_copy(y_sh.at[src1], rows1, sg1)
        pltpu.async_copy(src_hbm.at[pl.ds(ebase + 2 * CH, CH)], src2, ss2)
        pltpu.async_copy(dst_hbm.at[pl.ds(ebase + 2 * CH, CH)], dst2, sd2)
        pltpu.async_copy(src_hbm.at[pl.ds(ebase + 3 * CH, CH)], src3, ss3)
        pltpu.async_copy(dst_hbm.at[pl.ds(ebase + 3 * CH, CH)], dst3, sd3)

        def step(g, p):
            q = 1 - p
            a, b = 2 * p, 2 * p + 1
            e0, e1 = 2 * q, 2 * q + 1
            pltpu.make_async_copy(y_sh.at[src_b[a]], rows_b[a], sg[a]).wait()
            pltpu.make_async_copy(y_sh.at[src_b[b]], rows_b[b], sg[b]).wait()
            o2, o3 = idx_off(2 * g + 2), idx_off(2 * g + 3)
            pltpu.make_async_copy(src_hbm.at[pl.ds(o2, CH)], src_b[e0], ss[e0]).wait()
            pltpu.make_async_copy(dst_hbm.at[pl.ds(o2, CH)], dst_b[e0], sd[e0]).wait()
            pltpu.make_async_copy(src_hbm.at[pl.ds(o3, CH)], src_b[e1], ss[e1]).wait()
            pltpu.make_async_copy(dst_hbm.at[pl.ds(o3, CH)], dst_b[e1], sd[e1]).wait()
            pltpu.async_copy(y_sh.at[src_b[e0]], rows_b[e0], sg[e0])
            pltpu.async_copy(y_sh.at[src_b[e1]], rows_b[e1], sg[e1])
            pltpu.async_copy(rows_b[a], z_sh.at[dst_b[a]], sc0, add=True)
            pltpu.async_copy(rows_b[b], z_sh.at[dst_b[b]], sc1, add=True)
            pltpu.make_async_copy(rows_b[a], z_sh.at[dst_b[a]], sc0).wait()
            pltpu.make_async_copy(rows_b[b], z_sh.at[dst_b[b]], sc1).wait()
            o4, o5 = idx_off(2 * g + 4), idx_off(2 * g + 5)
            pltpu.async_copy(src_hbm.at[pl.ds(o4, CH)], src_b[a], ss[a])
            pltpu.async_copy(dst_hbm.at[pl.ds(o4, CH)], dst_b[a], sd[a])
            pltpu.async_copy(src_hbm.at[pl.ds(o5, CH)], src_b[b], ss[b])
            pltpu.async_copy(dst_hbm.at[pl.ds(o5, CH)], dst_b[b], sd[b])

        def ebody(gg, carry):
            step(2 * gg, 0)
            step(2 * gg + 1, 1)
            return carry

        lax.fori_loop(0, ng // 2, ebody, 0)
        p_last = (ng - 1) & 1
        q_last = 1 - p_last
        for i in (2 * q_last, 2 * q_last + 1):
            pltpu.make_async_copy(y_sh.at[src_b[i]], rows_b[i], sg[i]).wait()
        for i in (2 * p_last, 2 * p_last + 1):
            pltpu.make_async_copy(src_hbm.at[pl.ds(last, CH)], src_b[i], ss[i]).wait()
            pltpu.make_async_copy(dst_hbm.at[pl.ds(last, CH)], dst_b[i], sd[i]).wait()
        plsc.subcore_barrier()
        for t in range(rps // CH):
            rb = base + t * CH
            pltpu.sync_copy(dinv_hbm.at[pl.ds(rb, CH)], dbuf)
            pltpu.sync_copy(z_sh.at[pl.ds(rb, CH)], rows0)
            scale_rows(rows0)
            pltpu.sync_copy(rows0, out_hbm.at[pl.ds(c * np_ + rb, CH)])

            @pl.when(c == 1)
            def _():
                pltpu.sync_copy(y_sh.at[pl.ds(rb, CH)], rows1)
                scale_rows(rows1)
                pltpu.sync_copy(rows1, out_hbm.at[pl.ds(2 * np_ + rb, CH)])

    return pl.kernel(
        body,
        out_type=jax.ShapeDtypeStruct((3 * np_, h), jnp.float32),
        mesh=mesh,
        compiler_params=pltpu.CompilerParams(use_tc_tiling_on_sc=False),
        scratch_types=(
            [pltpu.VMEM((CH,), jnp.int32)] * 8
            + [pltpu.VMEM((CH, h), jnp.float32)] * 4
            + [pltpu.VMEM((zr, h), jnp.float32),
               pltpu.VMEM((CH,), jnp.float32),
               pltpu.VMEM_SHARED((np_, h), jnp.float32),
               pltpu.VMEM_SHARED((np_, h), jnp.float32)]
            + [pltpu.SemaphoreType.DMA] * 14
        ),
    )


@functools.lru_cache(None)
def _dinv_kernel(np_):
    def body(degp_ref, out_ref):
        deg = degp_ref[0, :] + degp_ref[1, :] + 1.0
        out_ref[...] = lax.rsqrt(deg)

    return pl.pallas_call(
        body, out_shape=jax.ShapeDtypeStruct((np_,), jnp.float32))


@functools.lru_cache(None)
def _first_tc(n, np_, f_in, h1):
    r = 1000

    def body(x_ref, w_ref, out_ref):
        out_ref[...] = jnp.dot(x_ref[...], w_ref[...],
                               preferred_element_type=jnp.float32,
                               precision=lax.Precision.HIGHEST)

    return pl.pallas_call(
        body,
        grid=(n // r,),
        in_specs=[
            pl.BlockSpec((r, f_in), lambda i: (i, 0)),
            pl.BlockSpec((f_in, h1), lambda i: (0, 0)),
        ],
        out_specs=pl.BlockSpec((r, h1), lambda i: (i, 0)),
        out_shape=jax.ShapeDtypeStruct((np_, h1), jnp.float32),
    )


@functools.lru_cache(None)
def _mid_tc(np_, hin, hout):
    r = 1024

    def body(z0_ref, z1_ref, y_ref, b_ref, w_ref, out_ref):
        agg = z0_ref[0] + z1_ref[0] + y_ref[0]
        hcur = jnp.maximum(agg + b_ref[...], 0.0)
        out_ref[...] = jnp.dot(hcur, w_ref[...],
                               preferred_element_type=jnp.float32,
                               precision=lax.Precision.HIGHEST)

    return pl.pallas_call(
        body,
        grid=(np_ // r,),
        in_specs=[
            pl.BlockSpec((1, r, hin), lambda i: (0, i, 0)),
            pl.BlockSpec((1, r, hin), lambda i: (1, i, 0)),
            pl.BlockSpec((1, r, hin), lambda i: (2, i, 0)),
            pl.BlockSpec((1, hin), lambda i: (0, 0)),
            pl.BlockSpec((hin, hout), lambda i: (0, 0)),
        ],
        out_specs=pl.BlockSpec((r, hout), lambda i: (i, 0)),
        out_shape=jax.ShapeDtypeStruct((np_, hout), jnp.float32),
    )


@functools.lru_cache(None)
def _final_tc(n, np_, hp, c_out):
    r = 1000

    def body(z0_ref, z1_ref, y_ref, b_ref, out_ref):
        res = z0_ref[0] + z1_ref[0] + y_ref[0] + b_ref[...]
        out_ref[...] = res[:, :c_out]

    return pl.pallas_call(
        body,
        grid=(n // r,),
        in_specs=[
            pl.BlockSpec((1, r, hp), lambda i: (0, i, 0)),
            pl.BlockSpec((1, r, hp), lambda i: (1, i, 0)),
            pl.BlockSpec((1, r, hp), lambda i: (2, i, 0)),
            pl.BlockSpec((1, hp), lambda i: (0, 0)),
        ],
        out_specs=pl.BlockSpec((r, c_out), lambda i: (i, 0)),
        out_shape=jax.ShapeDtypeStruct((n, c_out), jnp.float32),
    )


def kernel(x, edge_index, W1, b1, W2, b2, W3, b3):
    n, f_in = x.shape
    e = edge_index.shape[1]
    h1, h2, c_out = W1.shape[1], W2.shape[1], W3.shape[1]
    hp = -(-c_out // 16) * 16

    np_ = (n // 512 + 1) * 512
    ep = -(-e // (NW * CH * 4)) * (NW * CH * 4)

    src = edge_index[0]
    dst = edge_index[1]
    pad = ep - e
    if pad:
        fill = jnp.full((pad,), n, dtype=src.dtype)
        src = jnp.concatenate([src, fill])
        dst = jnp.concatenate([dst, fill])
    w3p = jnp.pad(W3, ((0, 0), (0, hp - c_out)))
    b3p = jnp.pad(b3, (0, hp - c_out)).reshape(1, hp)

    degp = _deg_kernel(np_, ep)(dst)
    dinv = _dinv_kernel(np_)(degp.reshape(NCORE, np_))

    u1 = _first_tc(n, np_, f_in, h1)(x, W1)
    z1 = _agg_kernel(np_, ep, h1)(u1, dinv, src, dst).reshape(3, np_, h1)
    y2 = _mid_tc(np_, h1, h2)(z1, z1, z1, b1.reshape(1, h1), W2)
    z2 = _agg_kernel(np_, ep, h2)(y2, dinv, src, dst).reshape(3, np_, h2)
    y3 = _mid_tc(np_, h2, hp)(z2, z2, z2, b2.reshape(1, h2), w3p)
    z3 = _agg_kernel(np_, ep, hp)(y3, dinv, src, dst).reshape(3, np_, hp)
    return _final_tc(n, np_, hp, c_out)(z3, z3, z3, b3p)


# speedup vs baseline: 1.0832x; 1.0117x over previous
import functools

import jax
import jax.numpy as jnp
from jax import lax
from jax.experimental import pallas as pl
from jax.experimental.pallas import tpu as pltpu
from jax.experimental.pallas import tpu_sc as plsc

NCORE = 2
NSUB = 16
NW = NCORE * NSUB
CH = 128


@functools.lru_cache(None)
def _deg_kernel(np_, ep):
    ew = ep // NW
    nch = ew // CH
    rps = np_ // NSUB
    zr = 64
    mesh = plsc.VectorSubcoreMesh(core_axis_name="c", subcore_axis_name="s")

    assert nch % 4 == 0
    ng = nch // 2

    def body(dst_hbm, out_hbm, dst0, dst1, dst2, dst3, ones_v, zb_v, z_sh,
             sd0, sd1, sd2, sd3, sc0, sc1):
        c = lax.axis_index("c")
        s = lax.axis_index("s")
        dst_b = (dst0, dst1, dst2, dst3)
        sd = (sd0, sd1, sd2, sd3)
        one16 = jnp.ones((16,), jnp.float32)
        zero16 = jnp.zeros((16,), jnp.float32)
        for j in range(CH // 16):
            ones_v[pl.ds(j * 16, 16)] = one16
        for j in range(zr // 16):
            zb_v[pl.ds(j * 16, 16)] = zero16
        base = s * rps

        def zbody(i, carry):
            pltpu.sync_copy(zb_v, z_sh.at[pl.ds(base + i * zr, zr)])
            return carry

        lax.fori_loop(0, rps // zr, zbody, 0)
        plsc.subcore_barrier()
        ebase = (c * NSUB + s) * ew
        last = ebase + (nch - 1) * CH

        def idx_off(m):
            return jnp.minimum(ebase + m * CH, last)

        pltpu.sync_copy(dst_hbm.at[pl.ds(ebase, CH)], dst0)
        pltpu.sync_copy(dst_hbm.at[pl.ds(ebase + CH, CH)], dst1)
        pltpu.async_copy(dst_hbm.at[pl.ds(ebase + 2 * CH, CH)], dst2, sd2)
        pltpu.async_copy(dst_hbm.at[pl.ds(ebase + 3 * CH, CH)], dst3, sd3)

        def step(g, p):
            q = 1 - p
            a, b = 2 * p, 2 * p + 1
            e0, e1 = 2 * q, 2 * q + 1
            o2, o3 = idx_off(2 * g + 2), idx_off(2 * g + 3)
            pltpu.make_async_copy(dst_hbm.at[pl.ds(o2, CH)], dst_b[e0], sd[e0]).wait()
            pltpu.make_async_copy(dst_hbm.at[pl.ds(o3, CH)], dst_b[e1], sd[e1]).wait()
            pltpu.async_copy(ones_v, z_sh.at[dst_b[a]], sc0, add=True)
            pltpu.async_copy(ones_v, z_sh.at[dst_b[b]], sc1, add=True)
            pltpu.make_async_copy(ones_v, z_sh.at[dst_b[a]], sc0).wait()
            pltpu.make_async_copy(ones_v, z_sh.at[dst_b[b]], sc1).wait()
            o4, o5 = idx_off(2 * g + 4), idx_off(2 * g + 5)
            pltpu.async_copy(dst_hbm.at[pl.ds(o4, CH)], dst_b[a], sd[a])
            pltpu.async_copy(dst_hbm.at[pl.ds(o5, CH)], dst_b[b], sd[b])

        def ebody(gg, carry):
            step(2 * gg, 0)
            step(2 * gg + 1, 1)
            return carry

        lax.fori_loop(0, ng // 2, ebody, 0)
        p_last = (ng - 1) & 1
        for i in (2 * p_last, 2 * p_last + 1):
            pltpu.make_async_copy(dst_hbm.at[pl.ds(last, CH)], dst_b[i], sd[i]).wait()
        plsc.subcore_barrier()
        pltpu.sync_copy(z_sh.at[pl.ds(base, rps)],
                        out_hbm.at[pl.ds(c * np_ + base, rps)])

    return pl.kernel(
        body,
        out_type=jax.ShapeDtypeStruct((NCORE * np_,), jnp.float32),
        mesh=mesh,
        compiler_params=pltpu.CompilerParams(use_tc_tiling_on_sc=False),
        scratch_types=(
            [pltpu.VMEM((CH,), jnp.int32)] * 4
            + [pltpu.VMEM((CH,), jnp.float32),
               pltpu.VMEM((zr,), jnp.float32),
               pltpu.VMEM_SHARED((np_,), jnp.float32)]
            + [pltpu.SemaphoreType.DMA] * 6
        ),
    )


@functools.lru_cache(None)
def _agg_kernel(np_, ep, h):
    ew = ep // NW
    nch = ew // CH
    assert nch % 4 == 0
    ng = nch // 2
    rps = np_ // NSUB
    zr = 128
    assert rps % zr == 0
    mesh = plsc.VectorSubcoreMesh(core_axis_name="c", subcore_axis_name="s")

    assert rps % CH == 0

    def body(y_hbm, dinv_hbm, src_hbm, dst_hbm, out_hbm,
             src0, src1, src2, src3, dst0, dst1, dst2, dst3,
             rows0, rows1, rows2, rows3, zb_v, dbuf, z_sh, y_sh,
             sg0, sg1, sg2, sg3, ss0, ss1, ss2, ss3,
             sd0, sd1, sd2, sd3, sc0, sc1):
        c = lax.axis_index("c")
        s = lax.axis_index("s")
        src_b = (src0, src1, src2, src3)
        dst_b = (dst0, dst1, dst2, dst3)
        rows_b = (rows0, rows1, rows2, rows3)
        sg = (sg0, sg1, sg2, sg3)
        ss = (ss0, ss1, ss2, ss3)
        sd = (sd0, sd1, sd2, sd3)
        zero16 = jnp.zeros((16,), jnp.float32)
        for r in range(zr):
            for j in range(h // 16):
                zb_v[r, pl.ds(j * 16, 16)] = zero16
        base = s * rps

        def scale_rows(buf):
            def rloop(i, carry):
                dv = dbuf[pl.ds(i * 16, 16)]
                for rr in range(16):
                    r = i * 16 + rr
                    sv = dv[rr]
                    for j in range(h // 16):
                        buf[r, pl.ds(16 * j, 16)] = buf[r, pl.ds(16 * j, 16)] * sv
                return carry
            lax.fori_loop(0, CH // 16, rloop, 0)

        def zbody(i, carry):
            pltpu.sync_copy(zb_v, z_sh.at[pl.ds(base + i * zr, zr)])
            return carry

        lax.fori_loop(0, rps // zr, zbody, 0)
        for t in range(rps // CH):
            rb = base + t * CH
            pltpu.sync_copy(y_hbm.at[pl.ds(rb, CH)], rows0)
            pltpu.sync_copy(dinv_hbm.at[pl.ds(rb, CH)], dbuf)
            scale_rows(rows0)
            pltpu.sync_copy(rows0, y_sh.at[pl.ds(rb, CH)])
        plsc.subcore_barrier()
        ebase = (c * NSUB + s) * ew
        last = ebase + (nch - 1) * CH

        def idx_off(m):
            return jnp.minimum(ebase + m * CH, last)

        pltpu.sync_copy(src_hbm.at[pl.ds(ebase, CH)], src0)
        pltpu.sync_copy(dst_hbm.at[pl.ds(ebase, CH)], dst0)
        pltpu.sync_copy(src_hbm.at[pl.ds(ebase + CH, CH)], src1)
        pltpu.sync_copy(dst_hbm.at[pl.ds(ebase + CH, CH)], dst1)
        pltpu.async_copy(y_sh.at[src0], rows0, sg0)
        pltpu.async_copy(y_sh.at[src1], rows1, sg1)
        pltpu.async_copy(src_hbm.at[pl.ds(ebase + 2 * CH, CH)], src2, ss2)
        pltpu.async_copy(dst_hbm.at[pl.ds(ebase + 2 * CH, CH)], dst2, sd2)
        pltpu.async_copy(src_hbm.at[pl.ds(ebase + 3 * CH, CH)], src3, ss3)
        pltpu.async_copy(dst_hbm.at[pl.ds(ebase + 3 * CH, CH)], dst3, sd3)

        def step(g, p):
            q = 1 - p
            a, b = 2 * p, 2 * p + 1
            e0, e1 = 2 * q, 2 * q + 1
            pltpu.make_async_copy(y_sh.at[src_b[a]], rows_b[a], sg[a]).wait()
            pltpu.make_async_copy(y_sh.at[src_b[b]], rows_b[b], sg[b]).wait()
            o2, o3 = idx_off(2 * g + 2), idx_off(2 * g + 3)
            pltpu.make_async_copy(src_hbm.at[pl.ds(o2, CH)], src_b[e0], ss[e0]).wait()
            pltpu.make_async_copy(dst_hbm.at[pl.ds(o2, CH)], dst_b[e0], sd[e0]).wait()
            pltpu.make_async_copy(src_hbm.at[pl.ds(o3, CH)], src_b[e1], ss[e1]).wait()
            pltpu.make_async_copy(dst_hbm.at[pl.ds(o3, CH)], dst_b[e1], sd[e1]).wait()
            pltpu.async_copy(y_sh.at[src_b[e0]], rows_b[e0], sg[e0])
            pltpu.async_copy(y_sh.at[src_b[e1]], rows_b[e1], sg[e1])
            pltpu.async_copy(rows_b[a], z_sh.at[dst_b[a]], sc0, add=True)
            pltpu.async_copy(rows_b[b], z_sh.at[dst_b[b]], sc1, add=True)
            pltpu.make_async_copy(rows_b[a], z_sh.at[dst_b[a]], sc0).wait()
            pltpu.make_async_copy(rows_b[b], z_sh.at[dst_b[b]], sc1).wait()
            o4, o5 = idx_off(2 * g + 4), idx_off(2 * g + 5)
            pltpu.async_copy(src_hbm.at[pl.ds(o4, CH)], src_b[a], ss[a])
            pltpu.async_copy(dst_hbm.at[pl.ds(o4, CH)], dst_b[a], sd[a])
            pltpu.async_copy(src_hbm.at[pl.ds(o5, CH)], src_b[b], ss[b])
            pltpu.async_copy(dst_hbm.at[pl.ds(o5, CH)], dst_b[b], sd[b])

        def ebody(gg, carry):
            step(2 * gg, 0)
            step(2 * gg + 1, 1)
            return carry

        lax.fori_loop(0, ng // 2, ebody, 0)
        p_last = (ng - 1) & 1
        q_last = 1 - p_last
        for i in (2 * q_last, 2 * q_last + 1):
            pltpu.make_async_copy(y_sh.at[src_b[i]], rows_b[i], sg[i]).wait()
        for i in (2 * p_last, 2 * p_last + 1):
            pltpu.make_async_copy(src_hbm.at[pl.ds(last, CH)], src_b[i], ss[i]).wait()
            pltpu.make_async_copy(dst_hbm.at[pl.ds(last, CH)], dst_b[i], sd[i]).wait()
        plsc.subcore_barrier()
        for t in range(rps // CH):
            rb = base + t * CH
            pltpu.sync_copy(dinv_hbm.at[pl.ds(rb, CH)], dbuf)
            pltpu.sync_copy(z_sh.at[pl.ds(rb, CH)], rows0)
            scale_rows(rows0)
            pltpu.sync_copy(rows0, out_hbm.at[pl.ds(c * np_ + rb, CH)])

            @pl.when((c == 0) == (s < NSUB // 2))
            def _():
                pltpu.sync_copy(y_sh.at[pl.ds(rb, CH)], rows1)
                scale_rows(rows1)
                pltpu.sync_copy(rows1, out_hbm.at[pl.ds(2 * np_ + rb, CH)])

    return pl.kernel(
        body,
        out_type=jax.ShapeDtypeStruct((3 * np_, h), jnp.float32),
        mesh=mesh,
        compiler_params=pltpu.CompilerParams(use_tc_tiling_on_sc=False),
        scratch_types=(
            [pltpu.VMEM((CH,), jnp.int32)] * 8
            + [pltpu.VMEM((CH, h), jnp.float32)] * 4
            + [pltpu.VMEM((zr, h), jnp.float32),
               pltpu.VMEM((CH,), jnp.float32),
               pltpu.VMEM_SHARED((np_, h), jnp.float32),
               pltpu.VMEM_SHARED((np_, h), jnp.float32)]
            + [pltpu.SemaphoreType.DMA] * 14
        ),
    )


@functools.lru_cache(None)
def _dinv_kernel(np_):
    def body(degp_ref, out_ref):
        deg = degp_ref[0, :] + degp_ref[1, :] + 1.0
        out_ref[...] = lax.rsqrt(deg)

    return pl.pallas_call(
        body, out_shape=jax.ShapeDtypeStruct((np_,), jnp.float32))


@functools.lru_cache(None)
def _first_tc(n, np_, f_in, h1):
    def body(x_ref, w_ref, out_ref):
        out_ref[pl.ds(0, n), :] = jnp.dot(x_ref[...], w_ref[...],
                                          preferred_element_type=jnp.float32,
                                          precision=lax.Precision.HIGHEST)

    return pl.pallas_call(
        body, out_shape=jax.ShapeDtypeStruct((np_, h1), jnp.float32))


@functools.lru_cache(None)
def _mid_tc(np_, hin, hout):
    def body(z_ref, b_ref, w_ref, out_ref):
        agg = z_ref[0] + z_ref[1] + z_ref[2]
        hcur = jnp.maximum(agg + b_ref[...], 0.0)
        out_ref[...] = jnp.dot(hcur, w_ref[...],
                               preferred_element_type=jnp.float32,
                               precision=lax.Precision.HIGHEST)

    return pl.pallas_call(
        body, out_shape=jax.ShapeDtypeStruct((np_, hout), jnp.float32))


@functools.lru_cache(None)
def _final_tc(n, np_, hp, c_out):
    def body(z_ref, b_ref, out_ref):
        res = (z_ref[0, pl.ds(0, n)] + z_ref[1, pl.ds(0, n)]
               + z_ref[2, pl.ds(0, n)] + b_ref[...])
        out_ref[...] = res[:, :c_out]

    return pl.pallas_call(
        body, out_shape=jax.ShapeDtypeStruct((n, c_out), jnp.float32))


def kernel(x, edge_index, W1, b1, W2, b2, W3, b3):
    n, f_in = x.shape
    e = edge_index.shape[1]
    h1, h2, c_out = W1.shape[1], W2.shape[1], W3.shape[1]
    hp = -(-c_out // 16) * 16

    np_ = (n // 512 + 1) * 512
    ep = -(-e // (NW * CH * 4)) * (NW * CH * 4)

    src = edge_index[0]
    dst = edge_index[1]
    pad = ep - e
    if pad:
        fill = jnp.full((pad,), n, dtype=src.dtype)
        src = jnp.concatenate([src, fill])
        dst = jnp.concatenate([dst, fill])
    w3p = jnp.pad(W3, ((0, 0), (0, hp - c_out)))
    b3p = jnp.pad(b3, (0, hp - c_out)).reshape(1, hp)

    degp = _deg_kernel(np_, ep)(dst)
    dinv = _dinv_kernel(np_)(degp.reshape(NCORE, np_))

    u1 = _first_tc(n, np_, f_in, h1)(x, W1)
    z1 = _agg_kernel(np_, ep, h1)(u1, dinv, src, dst).reshape(3, np_, h1)
    y2 = _mid_tc(np_, h1, h2)(z1, b1.reshape(1, h1), W2)
    z2 = _agg_kernel(np_, ep, h2)(y2, dinv, src, dst).reshape(3, np_, h2)
    y3 = _mid_tc(np_, h2, hp)(z2, b2.reshape(1, h2), w3p)
    z3 = _agg_kernel(np_, ep, hp)(y3, dinv, src, dst).reshape(3, np_, hp)
    return _final_tc(n, np_, hp, c_out)(z3, b3p)


# speedup vs baseline: 1.1427x; 1.0549x over previous
import functools

import jax
import jax.numpy as jnp
from jax import lax
from jax.experimental import pallas as pl
from jax.experimental.pallas import tpu as pltpu
from jax.experimental.pallas import tpu_sc as plsc

NCORE = 2
NSUB = 16
NW = NCORE * NSUB
CH = 128


@functools.lru_cache(None)
def _deg_kernel(np_, ep):
    ew = ep // NW
    nch = ew // CH
    rps = np_ // NSUB
    zr = 64
    mesh = plsc.VectorSubcoreMesh(core_axis_name="c", subcore_axis_name="s")

    assert nch % 4 == 0
    ng = nch // 2

    def body(dst_hbm, out_hbm, dst0, dst1, dst2, dst3, ones_v, zb_v, z_sh,
             sd0, sd1, sd2, sd3, sc0, sc1):
        c = lax.axis_index("c")
        s = lax.axis_index("s")
        dst_b = (dst0, dst1, dst2, dst3)
        sd = (sd0, sd1, sd2, sd3)
        one16 = jnp.ones((16,), jnp.float32)
        zero16 = jnp.zeros((16,), jnp.float32)
        for j in range(CH // 16):
            ones_v[pl.ds(j * 16, 16)] = one16
        for j in range(zr // 16):
            zb_v[pl.ds(j * 16, 16)] = zero16
        base = s * rps

        def zbody(i, carry):
            pltpu.sync_copy(zb_v, z_sh.at[pl.ds(base + i * zr, zr)])
            return carry

        lax.fori_loop(0, rps // zr, zbody, 0)
        plsc.subcore_barrier()
        ebase = (c * NSUB + s) * ew
        last = ebase + (nch - 1) * CH

        def idx_off(m):
            return jnp.minimum(ebase + m * CH, last)

        pltpu.sync_copy(dst_hbm.at[pl.ds(ebase, CH)], dst0)
        pltpu.sync_copy(dst_hbm.at[pl.ds(ebase + CH, CH)], dst1)
        pltpu.async_copy(dst_hbm.at[pl.ds(ebase + 2 * CH, CH)], dst2, sd2)
        pltpu.async_copy(dst_hbm.at[pl.ds(ebase + 3 * CH, CH)], dst3, sd3)

        def step(g, p):
            q = 1 - p
            a, b = 2 * p, 2 * p + 1
            e0, e1 = 2 * q, 2 * q + 1
            o2, o3 = idx_off(2 * g + 2), idx_off(2 * g + 3)
            pltpu.make_async_copy(dst_hbm.at[pl.ds(o2, CH)], dst_b[e0], sd[e0]).wait()
            pltpu.make_async_copy(dst_hbm.at[pl.ds(o3, CH)], dst_b[e1], sd[e1]).wait()
            pltpu.async_copy(ones_v, z_sh.at[dst_b[a]], sc0, add=True)
            pltpu.async_copy(ones_v, z_sh.at[dst_b[b]], sc1, add=True)
            pltpu.make_async_copy(ones_v, z_sh.at[dst_b[a]], sc0).wait()
            pltpu.make_async_copy(ones_v, z_sh.at[dst_b[b]], sc1).wait()
            o4, o5 = idx_off(2 * g + 4), idx_off(2 * g + 5)
            pltpu.async_copy(dst_hbm.at[pl.ds(o4, CH)], dst_b[a], sd[a])
            pltpu.async_copy(dst_hbm.at[pl.ds(o5, CH)], dst_b[b], sd[b])

        def ebody(gg, carry):
            step(2 * gg, 0)
            step(2 * gg + 1, 1)
            return carry

        lax.fori_loop(0, ng // 2, ebody, 0)
        p_last = (ng - 1) & 1
        for i in (2 * p_last, 2 * p_last + 1):
            pltpu.make_async_copy(dst_hbm.at[pl.ds(last, CH)], dst_b[i], sd[i]).wait()
        plsc.subcore_barrier()
        pltpu.sync_copy(z_sh.at[pl.ds(base, rps)],
                        out_hbm.at[pl.ds(c * np_ + base, rps)])

    return pl.kernel(
        body,
        out_type=jax.ShapeDtypeStruct((NCORE * np_,), jnp.float32),
        mesh=mesh,
        compiler_params=pltpu.CompilerParams(use_tc_tiling_on_sc=False),
        scratch_types=(
            [pltpu.VMEM((CH,), jnp.int32)] * 4
            + [pltpu.VMEM((CH,), jnp.float32),
               pltpu.VMEM((zr,), jnp.float32),
               pltpu.VMEM_SHARED((np_,), jnp.float32)]
            + [pltpu.SemaphoreType.DMA] * 6
        ),
    )


@functools.lru_cache(None)
def _agg_kernel(np_, ep, h):
    ew = ep // NW
    nch = ew // CH
    assert nch % 4 == 0
    ng = nch // 2
    rps = np_ // NSUB
    zr = 128
    assert rps % zr == 0
    mesh = plsc.VectorSubcoreMesh(core_axis_name="c", subcore_axis_name="s")

    assert rps % CH == 0

    def body(y_hbm, dinv_hbm, src_hbm, dst_hbm, out_hbm,
             src0, src1, src2, src3, dst0, dst1, dst2, dst3,
             rows0, rows1, rows2, rows3, zb_v, dbuf, z_sh, y_sh,
             sg0, sg1, sg2, sg3, ss0, ss1, ss2, ss3,
             sd0, sd1, sd2, sd3, sc0, sc1):
        c = lax.axis_index("c")
        s = lax.axis_index("s")
        src_b = (src0, src1, src2, src3)
        dst_b = (dst0, dst1, dst2, dst3)
        rows_b = (rows0, rows1, rows2, rows3)
        sg = (sg0, sg1, sg2, sg3)
        ss = (ss0, ss1, ss2, ss3)
        sd = (sd0, sd1, sd2, sd3)
        zero16 = jnp.zeros((16,), jnp.float32)
        for r in range(zr):
            for j in range(h // 16):
                zb_v[r, pl.ds(j * 16, 16)] = zero16
        base = s * rps

        def scale_rows(buf, t):
            def rloop(i, carry):
                dv = dbuf[pl.ds(t * CH + i * 16, 16)]
                for rr in range(16):
                    r = i * 16 + rr
                    sv = dv[rr]
                    for j in range(h // 16):
                        buf[r, pl.ds(16 * j, 16)] = buf[r, pl.ds(16 * j, 16)] * sv
                return carry
            lax.fori_loop(0, CH // 16, rloop, 0)

        def zbody(i, carry):
            pltpu.sync_copy(zb_v, z_sh.at[pl.ds(base + i * zr, zr)])
            return carry

        lax.fori_loop(0, rps // zr, zbody, 0)
        pltpu.sync_copy(dinv_hbm.at[pl.ds(base, rps)], dbuf)
        pltpu.async_copy(y_hbm.at[pl.ds(base, CH)], rows0, sg0)
        for t in range(rps // CH):
            cur, scur = (rows0, sg0) if t % 2 == 0 else (rows1, sg1)
            rb = base + t * CH
            pltpu.make_async_copy(y_hbm.at[pl.ds(rb, CH)], cur, scur).wait()
            if t + 1 < rps // CH:
                nxt, snxt = (rows1, sg1) if t % 2 == 0 else (rows0, sg0)
                pltpu.async_copy(y_hbm.at[pl.ds(rb + CH, CH)], nxt, snxt)
            scale_rows(cur, t)
            pltpu.sync_copy(cur, y_sh.at[pl.ds(rb, CH)])
        plsc.subcore_barrier()
        ebase = (c * NSUB + s) * ew
        last = ebase + (nch - 1) * CH

        def idx_off(m):
            return jnp.minimum(ebase + m * CH, last)

        pltpu.sync_copy(src_hbm.at[pl.ds(ebase, CH)], src0)
        pltpu.sync_copy(dst_hbm.at[pl.ds(ebase, CH)], dst0)
        pltpu.sync_copy(src_hbm.at[pl.ds(ebase + CH, CH)], src1)
        pltpu.sync_copy(dst_hbm.at[pl.ds(ebase + CH, CH)], dst1)
        pltpu.async_copy(y_sh.at[src0], rows0, sg0)
        pltpu.async_copy(y_sh.at[src1], rows1, sg1)
        pltpu.async_copy(src_hbm.at[pl.ds(ebase + 2 * CH, CH)], src2, ss2)
        pltpu.async_copy(dst_hbm.at[pl.ds(ebase + 2 * CH, CH)], dst2, sd2)
        pltpu.async_copy(src_hbm.at[pl.ds(ebase + 3 * CH, CH)], src3, ss3)
        pltpu.async_copy(dst_hbm.at[pl.ds(ebase + 3 * CH, CH)], dst3, sd3)

        def step(g, p):
            q = 1 - p
            a, b = 2 * p, 2 * p + 1
            e0, e1 = 2 * q, 2 * q + 1
            pltpu.make_async_copy(y_sh.at[src_b[a]], rows_b[a], sg[a]).wait()
            pltpu.make_async_copy(y_sh.at[src_b[b]], rows_b[b], sg[b]).wait()
            o2, o3 = idx_off(2 * g + 2), idx_off(2 * g + 3)
            pltpu.make_async_copy(src_hbm.at[pl.ds(o2, CH)], src_b[e0], ss[e0]).wait()
            pltpu.make_async_copy(dst_hbm.at[pl.ds(o2, CH)], dst_b[e0], sd[e0]).wait()
            pltpu.make_async_copy(src_hbm.at[pl.ds(o3, CH)], src_b[e1], ss[e1]).wait()
            pltpu.make_async_copy(dst_hbm.at[pl.ds(o3, CH)], dst_b[e1], sd[e1]).wait()
            pltpu.async_copy(y_sh.at[src_b[e0]], rows_b[e0], sg[e0])
            pltpu.async_copy(y_sh.at[src_b[e1]], rows_b[e1], sg[e1])
            pltpu.async_copy(rows_b[a], z_sh.at[dst_b[a]], sc0, add=True)
            pltpu.async_copy(rows_b[b], z_sh.at[dst_b[b]], sc1, add=True)
            pltpu.make_async_copy(rows_b[a], z_sh.at[dst_b[a]], sc0).wait()
            pltpu.make_async_copy(rows_b[b], z_sh.at[dst_b[b]], sc1).wait()
            o4, o5 = idx_off(2 * g + 4), idx_off(2 * g + 5)
            pltpu.async_copy(src_hbm.at[pl.ds(o4, CH)], src_b[a], ss[a])
            pltpu.async_copy(dst_hbm.at[pl.ds(o4, CH)], dst_b[a], sd[a])
            pltpu.async_copy(src_hbm.at[pl.ds(o5, CH)], src_b[b], ss[b])
            pltpu.async_copy(dst_hbm.at[pl.ds(o5, CH)], dst_b[b], sd[b])

        def ebody(gg, carry):
            step(2 * gg, 0)
            step(2 * gg + 1, 1)
            return carry

        lax.fori_loop(0, ng // 2, ebody, 0)
        p_last = (ng - 1) & 1
        q_last = 1 - p_last
        for i in (2 * q_last, 2 * q_last + 1):
            pltpu.make_async_copy(y_sh.at[src_b[i]], rows_b[i], sg[i]).wait()
        for i in (2 * p_last, 2 * p_last + 1):
            pltpu.make_async_copy(src_hbm.at[pl.ds(last, CH)], src_b[i], ss[i]).wait()
            pltpu.make_async_copy(dst_hbm.at[pl.ds(last, CH)], dst_b[i], sd[i]).wait()
        plsc.subcore_barrier()
        for t in range(rps // CH):
            rb = base + t * CH
            pltpu.sync_copy(z_sh.at[pl.ds(rb, CH)], rows0)
            scale_rows(rows0, t)
            pltpu.async_copy(rows0, out_hbm.at[pl.ds(c * np_ + rb, CH)], sg0)

            @pl.when((c == 0) == (s < NSUB // 2))
            def _():
                pltpu.sync_copy(y_sh.at[pl.ds(rb, CH)], rows1)
                scale_rows(rows1, t)
                pltpu.async_copy(rows1, out_hbm.at[pl.ds(2 * np_ + rb, CH)], sg1)
            pltpu.make_async_copy(rows0, out_hbm.at[pl.ds(c * np_ + rb, CH)], sg0).wait()

            @pl.when((c == 0) == (s < NSUB // 2))
            def _():
                pltpu.make_async_copy(rows1, out_hbm.at[pl.ds(2 * np_ + rb, CH)], sg1).wait()

    return pl.kernel(
        body,
        out_type=jax.ShapeDtypeStruct((3 * np_, h), jnp.float32),
        mesh=mesh,
        compiler_params=pltpu.CompilerParams(use_tc_tiling_on_sc=False),
        scratch_types=(
            [pltpu.VMEM((CH,), jnp.int32)] * 8
            + [pltpu.VMEM((CH, h), jnp.float32)] * 4
            + [pltpu.VMEM((zr, h), jnp.float32),
               pltpu.VMEM((rps,), jnp.float32),
               pltpu.VMEM_SHARED((np_, h), jnp.float32),
               pltpu.VMEM_SHARED((np_, h), jnp.float32)]
            + [pltpu.SemaphoreType.DMA] * 14
        ),
    )


@functools.lru_cache(None)
def _dinv_kernel(np_):
    def body(degp_ref, out_ref):
        deg = degp_ref[0, :] + degp_ref[1, :] + 1.0
        out_ref[...] = lax.rsqrt(deg)

    return pl.pallas_call(
        body, out_shape=jax.ShapeDtypeStruct((np_,), jnp.float32))


@functools.lru_cache(None)
def _first_tc(n, np_, f_in, h1):
    def body(x_ref, w_ref, out_ref):
        out_ref[pl.ds(0, n), :] = jnp.dot(x_ref[...], w_ref[...],
                                          preferred_element_type=jnp.float32,
                                          precision=lax.Precision.HIGHEST)

    return pl.pallas_call(
        body, out_shape=jax.ShapeDtypeStruct((np_, h1), jnp.float32))


@functools.lru_cache(None)
def _mid_tc(np_, hin, hout):
    def body(z_ref, b_ref, w_ref, out_ref):
        agg = z_ref[0] + z_ref[1] + z_ref[2]
        hcur = jnp.maximum(agg + b_ref[...], 0.0)
        out_ref[...] = jnp.dot(hcur, w_ref[...],
                               preferred_element_type=jnp.float32,
                               precision=lax.Precision.HIGHEST)

    return pl.pallas_call(
        body, out_shape=jax.ShapeDtypeStruct((np_, hout), jnp.float32))


@functools.lru_cache(None)
def _final_tc(n, np_, hp, c_out):
    def body(z_ref, b_ref, out_ref):
        res = (z_ref[0, pl.ds(0, n)] + z_ref[1, pl.ds(0, n)]
               + z_ref[2, pl.ds(0, n)] + b_ref[...])
        out_ref[...] = res[:, :c_out]

    return pl.pallas_call(
        body, out_shape=jax.ShapeDtypeStruct((n, c_out), jnp.float32))


def kernel(x, edge_index, W1, b1, W2, b2, W3, b3):
    n, f_in = x.shape
    e = edge_index.shape[1]
    h1, h2, c_out = W1.shape[1], W2.shape[1], W3.shape[1]
    hp = -(-c_out // 16) * 16

    np_ = (n // 512 + 1) * 512
    ep = -(-e // (NW * CH * 4)) * (NW * CH * 4)

    src = edge_index[0]
    dst = edge_index[1]
    pad = ep - e
    if pad:
        fill = jnp.full((pad,), n, dtype=src.dtype)
        src = jnp.concatenate([src, fill])
        dst = jnp.concatenate([dst, fill])
    w3p = jnp.pad(W3, ((0, 0), (0, hp - c_out)))
    b3p = jnp.pad(b3, (0, hp - c_out)).reshape(1, hp)

    degp = _deg_kernel(np_, ep)(dst)
    dinv = _dinv_kernel(np_)(degp.reshape(NCORE, np_))

    u1 = _first_tc(n, np_, f_in, h1)(x, W1)
    z1 = _agg_kernel(np_, ep, h1)(u1, dinv, src, dst).reshape(3, np_, h1)
    y2 = _mid_tc(np_, h1, h2)(z1, b1.reshape(1, h1), W2)
    z2 = _agg_kernel(np_, ep, h2)(y2, dinv, src, dst).reshape(3, np_, h2)
    y3 = _mid_tc(np_, h2, hp)(z2, b2.reshape(1, h2), w3p)
    z3 = _agg_kernel(np_, ep, hp)(y3, dinv, src, dst).reshape(3, np_, hp)
    return _final_tc(n, np_, hp, c_out)(z3, b3p)
